# Initial kernel scaffold; baseline (speedup 1.0000x reference)
import jax
import jax.numpy as jnp
from jax.experimental import pallas as pl


def kernel(x, edge_index, W1, att_src1, att_dst1, b1, W2, att_src2, att_dst2, b2):
    raise NotImplementedError("write your pallas kernel here")


# speedup vs baseline: 26.6164x; 26.6164x over previous
import functools

import jax
import jax.numpy as jnp
from jax import lax
from jax.experimental import pallas as pl
from jax.experimental.pallas import tpu as pltpu
from jax.experimental.pallas import tpu_sc as plsc

_N = 10000
_NP = 10240
_DIN = 128
_HID = 32
_NCLS = 40
_W = 48
_ONE = 40
_E = 320000
_CH = 128
_NTILES = 32
_CPT = 79
_EPT = _CPT * _CH
_EP = _NTILES * _EPT
_JUNK = 10016
_RPT = _NP // 16
_BLK = 512
_GRID = _NP // _BLK

_f32 = jnp.float32


def _lrelu_exp(a):
    return jnp.exp(jnp.maximum(a, 0.2 * a))


def _tc1_body(x_ref, w_ref, asr_ref, adr_ref, hp_ref, als_ref, ald_ref, ws_ref):
    h = jnp.dot(x_ref[...], w_ref[...], preferred_element_type=_f32)
    als = jnp.sum(h * asr_ref[...], axis=1, keepdims=True)
    ald = jnp.sum(h * adr_ref[...], axis=1, keepdims=True)
    b = h.shape[0]
    hp_ref[...] = jnp.concatenate(
        [h, jnp.zeros((b, _ONE - _HID), _f32), jnp.ones((b, 1), _f32),
         jnp.zeros((b, _W - _ONE - 1), _f32)], axis=1)
    als_ref[...] = als
    ald_ref[...] = ald
    ws_ref[...] = _lrelu_exp(als + ald)


def _tc2_body(a0_ref, a1_ref, hp1_ref, ws1_ref, b1_ref, w2_ref, asr_ref,
              adr_ref, hp_ref, als_ref, ald_ref, ws_ref):
    a0 = a0_ref[...]
    a1 = a1_ref[...]
    hp1 = hp1_ref[...]
    ws1 = ws1_ref[...]
    num = a0[:, :_HID] + a1[:, :_HID] + ws1 * hp1[:, :_HID]
    den = a0[:, _ONE:_ONE + 1] + a1[:, _ONE:_ONE + 1] + ws1 + 1e-16
    x2 = jax.nn.relu(num / den + b1_ref[...])
    h = jnp.dot(x2, w2_ref[...], preferred_element_type=_f32)
    als = jnp.sum(h * asr_ref[...], axis=1, keepdims=True)
    ald = jnp.sum(h * adr_ref[...], axis=1, keepdims=True)
    b = h.shape[0]
    hp_ref[...] = jnp.concatenate(
        [h, jnp.ones((b, 1), _f32), jnp.zeros((b, _W - _ONE - 1), _f32)],
        axis=1)
    als_ref[...] = als
    ald_ref[...] = ald
    ws_ref[...] = _lrelu_exp(als + ald)


def _tc3_body(a0_ref, a1_ref, hp2_ref, ws2_ref, b2_ref, out_ref):
    a0 = a0_ref[...]
    a1 = a1_ref[...]
    hp2 = hp2_ref[...]
    ws2 = ws2_ref[...]
    num = a0[:, :_NCLS] + a1[:, :_NCLS] + ws2 * hp2[:, :_NCLS]
    den = a0[:, _ONE:_ONE + 1] + a1[:, _ONE:_ONE + 1] + ws2 + 1e-16
    o = jax.nn.relu(num / den + b2_ref[...])
    m = jnp.max(o, axis=1, keepdims=True)
    e = o - m
    out_ref[...] = e - jnp.log(jnp.sum(jnp.exp(e), axis=1, keepdims=True))


def _row_block(width):
    return pl.BlockSpec((_BLK, width), lambda i: (i, 0))


def _full_block(shape):
    return pl.BlockSpec(shape, lambda i: (0,) * len(shape))


def _tc1(x, w1, asr, adr):
    return pl.pallas_call(
        _tc1_body,
        grid=(_GRID,),
        in_specs=[_row_block(_DIN), _full_block((_DIN, _HID)),
                  _full_block((1, _HID)), _full_block((1, _HID))],
        out_specs=[_row_block(_W), _row_block(1), _row_block(1),
                   _row_block(1)],
        out_shape=[jax.ShapeDtypeStruct((_NP, _W), _f32)] +
                  [jax.ShapeDtypeStruct((_NP, 1), _f32)] * 3,
    )(x, w1, asr, adr)


def _tc2(a0, a1, hp1, ws1, b1, w2, asr, adr):
    return pl.pallas_call(
        _tc2_body,
        grid=(_GRID,),
        in_specs=[_row_block(_W), _row_block(_W), _row_block(_W),
                  _row_block(1), _full_block((1, _HID)),
                  _full_block((_HID, _NCLS)), _full_block((1, _NCLS)),
                  _full_block((1, _NCLS))],
        out_specs=[_row_block(_W), _row_block(1), _row_block(1),
                   _row_block(1)],
        out_shape=[jax.ShapeDtypeStruct((_NP, _W), _f32)] +
                  [jax.ShapeDtypeStruct((_NP, 1), _f32)] * 3,
    )(a0, a1, hp1, ws1, b1, w2, asr, adr)


def _tc3(a0, a1, hp2, ws2, b2):
    return pl.pallas_call(
        _tc3_body,
        grid=(_GRID,),
        in_specs=[_row_block(_W), _row_block(_W), _row_block(_W),
                  _row_block(1), _full_block((1, _NCLS))],
        out_specs=_row_block(_NCLS),
        out_shape=jax.ShapeDtypeStruct((_NP, _NCLS), _f32),
    )(a0, a1, hp2, ws2, b2)


def _sc_body(src_hbm, dst_hbm, as_hbm, ad_hbm, hp_hbm, out_hbm,
             as_v, ad_v, schunk, dchunk, rows, wbuf, zbuf, acc, sem):
    c = lax.axis_index("c")
    s = lax.axis_index("s")
    wid = s * 2 + c

    if True:
        pltpu.sync_copy(as_hbm, as_v)
        pltpu.sync_copy(ad_hbm, ad_v)
        base = wid * _EPT

        def zb(i, carry):
            for j in range(3):
                zbuf[i, pl.ds(j * 16, 16)] = jnp.zeros((16,), _f32)
            return carry
        lax.fori_loop(0, 64, zb, 0)

        def za(i, carry):
            pltpu.sync_copy(zbuf, acc.at[pl.ds(s * _RPT + i * 64, 64)])
            return carry
        lax.fori_loop(0, _RPT // 64, za, 0)
        plsc.subcore_barrier()

        def chunk(g, carry):
            off = base + g * _CH
            pltpu.sync_copy(src_hbm.at[pl.ds(off, _CH)], schunk)
            pltpu.sync_copy(dst_hbm.at[pl.ds(off, _CH)], dchunk)
            pltpu.async_copy(hp_hbm.at[schunk], rows, sem).wait()

            def wk(kk, cy):
                si = schunk[pl.ds(kk * 16, 16)]
                di = dchunk[pl.ds(kk * 16, 16)]
                a = (plsc.load_gather(as_v, [si]) +
                     plsc.load_gather(ad_v, [di]))
                wbuf[pl.ds(kk * 16, 16)] = jnp.exp(jnp.maximum(a, 0.2 * a))
                return cy
            lax.fori_loop(0, _CH // 16, wk, 0)

            def scale(k, cy):
                wv = plsc.load_gather(wbuf, [jnp.broadcast_to(k, (16,))])
                for j in range(3):
                    rows[k, pl.ds(j * 16, 16)] = rows[k, pl.ds(j * 16, 16)] * wv
                return cy
            lax.fori_loop(0, _CH, scale, 0)

            pltpu.sync_copy(rows, acc.at[dchunk], add=True)
            return carry
        lax.fori_loop(0, _CPT, chunk, 0)
        plsc.subcore_barrier()

        def cp(i, carry):
            r0 = s * _RPT + i * 64
            pltpu.sync_copy(acc.at[pl.ds(r0, 64)],
                            out_hbm.at[pl.ds(c * _NP + r0, 64)])
            return carry
        lax.fori_loop(0, _RPT // 64, cp, 0)


@functools.lru_cache(maxsize=None)
def _make_sc_agg():
    @functools.partial(
        pl.kernel,
        mesh=plsc.VectorSubcoreMesh(core_axis_name="c", subcore_axis_name="s"),
        out_type=jax.ShapeDtypeStruct((2 * _NP, _W), _f32),
        compiler_params=pltpu.CompilerParams(needs_layout_passes=False,
                                             use_tc_tiling_on_sc=False),
        scratch_types=[
            pltpu.VMEM((_NP,), _f32),
            pltpu.VMEM((_NP,), _f32),
            pltpu.VMEM((_CH,), jnp.int32),
            pltpu.VMEM((_CH,), jnp.int32),
            pltpu.VMEM((_CH, _W), _f32),
            pltpu.VMEM((_CH,), _f32),
            pltpu.VMEM((64, _W), _f32),
            pltpu.VMEM_SHARED((_NP, _W), _f32),
            pltpu.SemaphoreType.DMA,
        ],
    )
    def _sc_agg(src_hbm, dst_hbm, as_hbm, ad_hbm, hp_hbm, out_hbm, *scratch):
        _sc_body(src_hbm, dst_hbm, as_hbm, ad_hbm, hp_hbm, out_hbm, *scratch)

    return _sc_agg


def kernel(x, edge_index, W1, att_src1, att_dst1, b1, W2, att_src2, att_dst2,
           b2):
    xp = jnp.zeros((_NP, _DIN), _f32).at[:_N].set(x)
    src = jnp.full((_EP,), 0, jnp.int32).at[:_E].set(
        edge_index[0].astype(jnp.int32))
    dst = jnp.full((_EP,), _JUNK, jnp.int32).at[:_E].set(
        edge_index[1].astype(jnp.int32))

    asr1 = att_src1.reshape(1, _HID)
    adr1 = att_dst1.reshape(1, _HID)
    asr2 = att_src2.reshape(1, _NCLS)
    adr2 = att_dst2.reshape(1, _NCLS)

    sc_agg = _make_sc_agg()
    hp1, als1, ald1, ws1 = _tc1(xp, W1, asr1, adr1)
    acc1 = sc_agg(src, dst, als1.reshape(_NP), ald1.reshape(_NP), hp1)
    hp2, als2, ald2, ws2 = _tc2(acc1[:_NP], acc1[_NP:], hp1, ws1,
                                b1.reshape(1, _HID), W2, asr2, adr2)
    acc2 = sc_agg(src, dst, als2.reshape(_NP), ald2.reshape(_NP), hp2)
    out = _tc3(acc2[:_NP], acc2[_NP:], hp2, ws2, b2.reshape(1, _NCLS))
    return out[:_N]


# speedup vs baseline: 29.4790x; 1.1076x over previous
import functools

import jax
import jax.numpy as jnp
from jax import lax
from jax.experimental import pallas as pl
from jax.experimental.pallas import tpu as pltpu
from jax.experimental.pallas import tpu_sc as plsc

_N = 10000
_NP = 10240
_DIN = 128
_HID = 32
_NCLS = 40
_W = 48
_ONE = 40
_E = 320000
_CH = 128
_NTILES = 32
_CPT = 80
_EPT = _CPT * _CH
_EP = _NTILES * _EPT
_JUNK = 10016
_RPT = _NP // 16
_BLK = 512
_GRID = _NP // _BLK

_f32 = jnp.float32


def _lrelu_exp(a):
    return jnp.exp(jnp.maximum(a, 0.2 * a))


def _tc1_body(x_ref, w_ref, asr_ref, adr_ref, hp_ref, als_ref, ald_ref, ws_ref):
    h = jnp.dot(x_ref[...], w_ref[...], preferred_element_type=_f32)
    als = jnp.sum(h * asr_ref[...], axis=1, keepdims=True)
    ald = jnp.sum(h * adr_ref[...], axis=1, keepdims=True)
    b = h.shape[0]
    hp_ref[...] = jnp.concatenate(
        [h, jnp.zeros((b, _ONE - _HID), _f32), jnp.ones((b, 1), _f32),
         jnp.zeros((b, _W - _ONE - 1), _f32)], axis=1)
    als_ref[...] = als
    ald_ref[...] = ald
    ws_ref[...] = _lrelu_exp(als + ald)


def _tc2_body(a0_ref, a1_ref, hp1_ref, ws1_ref, b1_ref, w2_ref, asr_ref,
              adr_ref, hp_ref, als_ref, ald_ref, ws_ref):
    a0 = a0_ref[...]
    a1 = a1_ref[...]
    hp1 = hp1_ref[...]
    ws1 = ws1_ref[...]
    num = a0[:, :_HID] + a1[:, :_HID] + ws1 * hp1[:, :_HID]
    den = a0[:, _ONE:_ONE + 1] + a1[:, _ONE:_ONE + 1] + ws1 + 1e-16
    x2 = jax.nn.relu(num / den + b1_ref[...])
    h = jnp.dot(x2, w2_ref[...], preferred_element_type=_f32)
    als = jnp.sum(h * asr_ref[...], axis=1, keepdims=True)
    ald = jnp.sum(h * adr_ref[...], axis=1, keepdims=True)
    b = h.shape[0]
    hp_ref[...] = jnp.concatenate(
        [h, jnp.ones((b, 1), _f32), jnp.zeros((b, _W - _ONE - 1), _f32)],
        axis=1)
    als_ref[...] = als
    ald_ref[...] = ald
    ws_ref[...] = _lrelu_exp(als + ald)


def _tc3_body(a0_ref, a1_ref, hp2_ref, ws2_ref, b2_ref, out_ref):
    a0 = a0_ref[...]
    a1 = a1_ref[...]
    hp2 = hp2_ref[...]
    ws2 = ws2_ref[...]
    num = a0[:, :_NCLS] + a1[:, :_NCLS] + ws2 * hp2[:, :_NCLS]
    den = a0[:, _ONE:_ONE + 1] + a1[:, _ONE:_ONE + 1] + ws2 + 1e-16
    o = jax.nn.relu(num / den + b2_ref[...])
    m = jnp.max(o, axis=1, keepdims=True)
    e = o - m
    out_ref[...] = e - jnp.log(jnp.sum(jnp.exp(e), axis=1, keepdims=True))


def _row_block(width):
    return pl.BlockSpec((_BLK, width), lambda i: (i, 0))


def _full_block(shape):
    return pl.BlockSpec(shape, lambda i: (0,) * len(shape))


def _tc1(x, w1, asr, adr):
    return pl.pallas_call(
        _tc1_body,
        grid=(_GRID,),
        in_specs=[_row_block(_DIN), _full_block((_DIN, _HID)),
                  _full_block((1, _HID)), _full_block((1, _HID))],
        out_specs=[_row_block(_W), _row_block(1), _row_block(1),
                   _row_block(1)],
        out_shape=[jax.ShapeDtypeStruct((_NP, _W), _f32)] +
                  [jax.ShapeDtypeStruct((_NP, 1), _f32)] * 3,
    )(x, w1, asr, adr)


def _tc2(a0, a1, hp1, ws1, b1, w2, asr, adr):
    return pl.pallas_call(
        _tc2_body,
        grid=(_GRID,),
        in_specs=[_row_block(_W), _row_block(_W), _row_block(_W),
                  _row_block(1), _full_block((1, _HID)),
                  _full_block((_HID, _NCLS)), _full_block((1, _NCLS)),
                  _full_block((1, _NCLS))],
        out_specs=[_row_block(_W), _row_block(1), _row_block(1),
                   _row_block(1)],
        out_shape=[jax.ShapeDtypeStruct((_NP, _W), _f32)] +
                  [jax.ShapeDtypeStruct((_NP, 1), _f32)] * 3,
    )(a0, a1, hp1, ws1, b1, w2, asr, adr)


def _tc3(a0, a1, hp2, ws2, b2):
    return pl.pallas_call(
        _tc3_body,
        grid=(_GRID,),
        in_specs=[_row_block(_W), _row_block(_W), _row_block(_W),
                  _row_block(1), _full_block((1, _NCLS))],
        out_specs=_row_block(_NCLS),
        out_shape=jax.ShapeDtypeStruct((_NP, _NCLS), _f32),
    )(a0, a1, hp2, ws2, b2)


def _sc_body(ebuf_hbm, as_hbm, ad_hbm, hp_hbm, out_hbm,
             as_v, ad_v, eb0, eb1, db0, db1, rows0, rows1, wbuf, zbuf, acc,
             semi0, semi1, semg0, semg1, sems0, sems1):
    c = lax.axis_index("c")
    s = lax.axis_index("s")
    wid = s * 2 + c
    cbase = wid * _CPT
    eb = (eb0, eb1)
    db = (db0, db1)
    rows = (rows0, rows1)
    semi = (semi0, semi1)
    semg = (semg0, semg1)
    sems = (sems0, sems1)

    pltpu.sync_copy(as_hbm, as_v)
    pltpu.sync_copy(ad_hbm, ad_v)

    def zb(i, carry):
        for j in range(3):
            zbuf[i, pl.ds(j * 16, 16)] = jnp.zeros((16,), _f32)
        return carry
    lax.fori_loop(0, 64, zb, 0)

    def za(i, carry):
        pltpu.sync_copy(zbuf, acc.at[pl.ds(s * _RPT + i * 64, 64)])
        return carry
    lax.fori_loop(0, _RPT // 64, za, 0)
    plsc.subcore_barrier()

    pltpu.async_copy(ebuf_hbm.at[cbase], eb0, semi0)
    pltpu.async_copy(ebuf_hbm.at[cbase + 1], eb1, semi1)
    pltpu.make_async_copy(ebuf_hbm.at[cbase], eb0, semi0).wait()
    pltpu.async_copy(hp_hbm.at[eb0.at[0]], rows0, semg0)

    def pair(g2, carry):
        for b in range(2):
            g = g2 * 2 + b
            nb = 1 - b
            pltpu.make_async_copy(hp_hbm.at[eb[b].at[0]], rows[b],
                                  semg[b]).wait()
            for kk in range(_CH // 16):
                si = eb[b][0, pl.ds(kk * 16, 16)]
                di = eb[b][1, pl.ds(kk * 16, 16)]
                db[b][pl.ds(kk * 16, 16)] = di
                a = (plsc.load_gather(as_v, [si]) +
                     plsc.load_gather(ad_v, [di]))
                wbuf[pl.ds(kk * 16, 16)] = jnp.exp(jnp.maximum(a, 0.2 * a))
            def scale(k, cy):
                wv = plsc.load_gather(wbuf, [jnp.broadcast_to(k, (16,))])
                for j in range(3):
                    rows[b][k, pl.ds(j * 16, 16)] = (
                        rows[b][k, pl.ds(j * 16, 16)] * wv)
                return cy
            lax.fori_loop(0, _CH, scale, 0, unroll=4)
            pltpu.async_copy(rows[b], acc.at[db[b]], sems[b], add=True)
            @pl.when(g + 2 < _CPT)
            def _():
                pltpu.async_copy(ebuf_hbm.at[cbase + g + 2], eb[b], semi[b])
            @pl.when(g >= 1)
            def _():
                pltpu.make_async_copy(rows[nb], acc.at[db[nb]],
                                      sems[nb]).wait()
            @pl.when(g + 1 < _CPT)
            def _():
                pltpu.make_async_copy(ebuf_hbm.at[cbase + g + 1], eb[nb],
                                      semi[nb]).wait()
                pltpu.async_copy(hp_hbm.at[eb[nb].at[0]], rows[nb], semg[nb])
        return carry
    lax.fori_loop(0, _CPT // 2, pair, 0)
    pltpu.make_async_copy(rows[(_CPT - 1) % 2], acc.at[db[(_CPT - 1) % 2]],
                          sems[(_CPT - 1) % 2]).wait()
    plsc.subcore_barrier()

    def cp(i, carry):
        r0 = s * _RPT + i * 64
        pltpu.sync_copy(acc.at[pl.ds(r0, 64)],
                        out_hbm.at[pl.ds(c * _NP + r0, 64)])
        return carry
    lax.fori_loop(0, _RPT // 64, cp, 0)


@functools.lru_cache(maxsize=None)
def _make_sc_agg():
    @functools.partial(
        pl.kernel,
        mesh=plsc.VectorSubcoreMesh(core_axis_name="c", subcore_axis_name="s"),
        out_type=jax.ShapeDtypeStruct((2 * _NP, _W), _f32),
        compiler_params=pltpu.CompilerParams(needs_layout_passes=False,
                                             use_tc_tiling_on_sc=False),
        scratch_types=[
            pltpu.VMEM((_NP,), _f32),
            pltpu.VMEM((_NP,), _f32),
            pltpu.VMEM((2, _CH), jnp.int32),
            pltpu.VMEM((2, _CH), jnp.int32),
            pltpu.VMEM((_CH,), jnp.int32),
            pltpu.VMEM((_CH,), jnp.int32),
            pltpu.VMEM((_CH, _W), _f32),
            pltpu.VMEM((_CH, _W), _f32),
            pltpu.VMEM((_CH,), _f32),
            pltpu.VMEM((64, _W), _f32),
            pltpu.VMEM_SHARED((_NP, _W), _f32),
            pltpu.SemaphoreType.DMA,
            pltpu.SemaphoreType.DMA,
            pltpu.SemaphoreType.DMA,
            pltpu.SemaphoreType.DMA,
            pltpu.SemaphoreType.DMA,
            pltpu.SemaphoreType.DMA,
        ],
    )
    def _sc_agg(ebuf_hbm, as_hbm, ad_hbm, hp_hbm, out_hbm, *scratch):
        _sc_body(ebuf_hbm, as_hbm, ad_hbm, hp_hbm, out_hbm, *scratch)

    return _sc_agg


def kernel(x, edge_index, W1, att_src1, att_dst1, b1, W2, att_src2, att_dst2,
           b2):
    xp = jnp.zeros((_NP, _DIN), _f32).at[:_N].set(x)
    src = jnp.full((_EP,), 0, jnp.int32).at[:_E].set(
        edge_index[0].astype(jnp.int32))
    dst = jnp.full((_EP,), _JUNK, jnp.int32).at[:_E].set(
        edge_index[1].astype(jnp.int32))
    ebuf = jnp.stack([src.reshape(_NTILES * _CPT, _CH),
                      dst.reshape(_NTILES * _CPT, _CH)], axis=1)

    asr1 = att_src1.reshape(1, _HID)
    adr1 = att_dst1.reshape(1, _HID)
    asr2 = att_src2.reshape(1, _NCLS)
    adr2 = att_dst2.reshape(1, _NCLS)

    sc_agg = _make_sc_agg()
    hp1, als1, ald1, ws1 = _tc1(xp, W1, asr1, adr1)
    acc1 = sc_agg(ebuf, als1.reshape(_NP), ald1.reshape(_NP), hp1)
    hp2, als2, ald2, ws2 = _tc2(acc1[:_NP], acc1[_NP:], hp1, ws1,
                                b1.reshape(1, _HID), W2, asr2, adr2)
    acc2 = sc_agg(ebuf, als2.reshape(_NP), ald2.reshape(_NP), hp2)
    out = _tc3(acc2[:_NP], acc2[_NP:], hp2, ws2, b2.reshape(1, _NCLS))
    return out[:_N]


# speedup vs baseline: 29.7470x; 1.0091x over previous
import functools

import jax
import jax.numpy as jnp
from jax import lax
from jax.experimental import pallas as pl
from jax.experimental.pallas import tpu as pltpu
from jax.experimental.pallas import tpu_sc as plsc

_N = 10000
_NP = 10240
_DIN = 128
_HID = 32
_NCLS = 40
_W1 = 48
_ONE1 = 32
_W2 = 48
_ONE2 = 40
_E = 320000
_CH = 128
_NTILES = 32
_CPT = 80
_EPT = _CPT * _CH
_EP = _NTILES * _EPT
_JUNK = 10016
_RPT = _NP // 16
_BLK = 512
_GRID = _NP // _BLK

_f32 = jnp.float32


def _lrelu_exp(a):
    return jnp.exp(jnp.maximum(a, 0.2 * a))


def _tc1_body(x_ref, w_ref, asr_ref, adr_ref, hp_ref, als_ref, ald_ref, ws_ref):
    h = jnp.dot(x_ref[...], w_ref[...], preferred_element_type=_f32)
    als = jnp.sum(h * asr_ref[...], axis=1, keepdims=True)
    ald = jnp.sum(h * adr_ref[...], axis=1, keepdims=True)
    b = h.shape[0]
    hp_ref[...] = jnp.concatenate(
        [h, jnp.ones((b, 1), _f32), jnp.zeros((b, _W1 - _ONE1 - 1), _f32)],
        axis=1)
    als_ref[...] = als
    ald_ref[...] = ald
    ws_ref[...] = _lrelu_exp(als + ald)


def _tc2_body(a0_ref, a1_ref, hp1_ref, ws1_ref, b1_ref, w2_ref, asr_ref,
              adr_ref, hp_ref, als_ref, ald_ref, ws_ref):
    a0 = a0_ref[...]
    a1 = a1_ref[...]
    hp1 = hp1_ref[...]
    ws1 = ws1_ref[...]
    num = a0[:, :_HID] + a1[:, :_HID] + ws1 * hp1[:, :_HID]
    den = a0[:, _ONE1:_ONE1 + 1] + a1[:, _ONE1:_ONE1 + 1] + ws1 + 1e-16
    x2 = jax.nn.relu(num / den + b1_ref[...])
    h = jnp.dot(x2, w2_ref[...], preferred_element_type=_f32)
    als = jnp.sum(h * asr_ref[...], axis=1, keepdims=True)
    ald = jnp.sum(h * adr_ref[...], axis=1, keepdims=True)
    b = h.shape[0]
    hp_ref[...] = jnp.concatenate(
        [h, jnp.ones((b, 1), _f32), jnp.zeros((b, _W2 - _ONE2 - 1), _f32)],
        axis=1)
    als_ref[...] = als
    ald_ref[...] = ald
    ws_ref[...] = _lrelu_exp(als + ald)


def _tc3_body(a0_ref, a1_ref, hp2_ref, ws2_ref, b2_ref, out_ref):
    a0 = a0_ref[...]
    a1 = a1_ref[...]
    hp2 = hp2_ref[...]
    ws2 = ws2_ref[...]
    num = a0[:, :_NCLS] + a1[:, :_NCLS] + ws2 * hp2[:, :_NCLS]
    den = a0[:, _ONE2:_ONE2 + 1] + a1[:, _ONE2:_ONE2 + 1] + ws2 + 1e-16
    o = jax.nn.relu(num / den + b2_ref[...])
    m = jnp.max(o, axis=1, keepdims=True)
    e = o - m
    out_ref[...] = e - jnp.log(jnp.sum(jnp.exp(e), axis=1, keepdims=True))


def _row_block(width):
    return pl.BlockSpec((_BLK, width), lambda i: (i, 0))


def _full_block(shape):
    return pl.BlockSpec(shape, lambda i: (0,) * len(shape))


def _tc1(x, w1, asr, adr):
    return pl.pallas_call(
        _tc1_body,
        grid=(_GRID,),
        in_specs=[_row_block(_DIN), _full_block((_DIN, _HID)),
                  _full_block((1, _HID)), _full_block((1, _HID))],
        out_specs=[_row_block(_W1), _row_block(1), _row_block(1),
                   _row_block(1)],
        out_shape=[jax.ShapeDtypeStruct((_NP, _W1), _f32)] +
                  [jax.ShapeDtypeStruct((_NP, 1), _f32)] * 3,
    )(x, w1, asr, adr)


def _tc2(a0, a1, hp1, ws1, b1, w2, asr, adr):
    return pl.pallas_call(
        _tc2_body,
        grid=(_GRID,),
        in_specs=[_row_block(_W1), _row_block(_W1), _row_block(_W1),
                  _row_block(1), _full_block((1, _HID)),
                  _full_block((_HID, _NCLS)), _full_block((1, _NCLS)),
                  _full_block((1, _NCLS))],
        out_specs=[_row_block(_W2), _row_block(1), _row_block(1),
                   _row_block(1)],
        out_shape=[jax.ShapeDtypeStruct((_NP, _W2), _f32)] +
                  [jax.ShapeDtypeStruct((_NP, 1), _f32)] * 3,
    )(a0, a1, hp1, ws1, b1, w2, asr, adr)


def _tc3(a0, a1, hp2, ws2, b2):
    return pl.pallas_call(
        _tc3_body,
        grid=(_GRID,),
        in_specs=[_row_block(_W2), _row_block(_W2), _row_block(_W2),
                  _row_block(1), _full_block((1, _NCLS))],
        out_specs=_row_block(_NCLS),
        out_shape=jax.ShapeDtypeStruct((_NP, _NCLS), _f32),
    )(a0, a1, hp2, ws2, b2)


def _sc_body(width, ebuf_hbm, as_hbm, ad_hbm, hp_hbm, out_hbm,
             as_v, ad_v, eb0, eb1, db0, db1, rows0, rows1, wbuf, zbuf, acc,
             semi0, semi1, semg0, semg1, sems0, sems1):
    c = lax.axis_index("c")
    s = lax.axis_index("s")
    wid = s * 2 + c
    cbase = wid * _CPT
    eb = (eb0, eb1)
    db = (db0, db1)
    rows = (rows0, rows1)
    semi = (semi0, semi1)
    semg = (semg0, semg1)
    sems = (sems0, sems1)

    pltpu.sync_copy(as_hbm, as_v)
    pltpu.sync_copy(ad_hbm, ad_v)

    zoffs = [j * 16 for j in range(width // 16)]
    if width % 16:
        zoffs.append(width - 16)

    def zb(i, carry):
        for o in zoffs:
            zbuf[i, pl.ds(o, 16)] = jnp.zeros((16,), _f32)
        return carry
    lax.fori_loop(0, 64, zb, 0)

    def za(i, carry):
        pltpu.sync_copy(zbuf, acc.at[pl.ds(s * _RPT + i * 64, 64)])
        return carry
    lax.fori_loop(0, _RPT // 64, za, 0)
    plsc.subcore_barrier()

    pltpu.async_copy(ebuf_hbm.at[cbase], eb0, semi0)
    pltpu.async_copy(ebuf_hbm.at[cbase + 1], eb1, semi1)
    pltpu.make_async_copy(ebuf_hbm.at[cbase], eb0, semi0).wait()
    pltpu.async_copy(hp_hbm.at[eb0.at[0]], rows0, semg0)

    def pair(g2, carry):
        for b in range(2):
            g = g2 * 2 + b
            nb = 1 - b
            pltpu.make_async_copy(hp_hbm.at[eb[b].at[0]], rows[b],
                                  semg[b]).wait()
            for kk in range(_CH // 16):
                si = eb[b][0, pl.ds(kk * 16, 16)]
                di = eb[b][1, pl.ds(kk * 16, 16)]
                db[b][pl.ds(kk * 16, 16)] = di
                a = (plsc.load_gather(as_v, [si]) +
                     plsc.load_gather(ad_v, [di]))
                wbuf[pl.ds(kk * 16, 16)] = jnp.exp(jnp.maximum(a, 0.2 * a))
            nfull = width // 16
            tail = width % 16
            tail_mask = lax.iota(jnp.int32, 16) < (16 - tail)

            def scale(k, cy):
                wv = plsc.load_gather(wbuf, [jnp.broadcast_to(k, (16,))])
                for j in range(nfull):
                    rows[b][k, pl.ds(j * 16, 16)] = (
                        rows[b][k, pl.ds(j * 16, 16)] * wv)
                if tail:
                    wt = jnp.where(tail_mask, jnp.ones((16,), _f32), wv)
                    rows[b][k, pl.ds(width - 16, 16)] = (
                        rows[b][k, pl.ds(width - 16, 16)] * wt)
                return cy
            lax.fori_loop(0, _CH, scale, 0, unroll=4)
            pltpu.async_copy(rows[b], acc.at[db[b]], sems[b], add=True)
            @pl.when(g + 2 < _CPT)
            def _():
                pltpu.async_copy(ebuf_hbm.at[cbase + g + 2], eb[b], semi[b])
            @pl.when(g >= 1)
            def _():
                pltpu.make_async_copy(rows[nb], acc.at[db[nb]],
                                      sems[nb]).wait()
            @pl.when(g + 1 < _CPT)
            def _():
                pltpu.make_async_copy(ebuf_hbm.at[cbase + g + 1], eb[nb],
                                      semi[nb]).wait()
                pltpu.async_copy(hp_hbm.at[eb[nb].at[0]], rows[nb], semg[nb])
        return carry
    lax.fori_loop(0, _CPT // 2, pair, 0)
    pltpu.make_async_copy(rows[(_CPT - 1) % 2], acc.at[db[(_CPT - 1) % 2]],
                          sems[(_CPT - 1) % 2]).wait()
    plsc.subcore_barrier()

    def cp(i, carry):
        r0 = s * _RPT + i * 64
        pltpu.sync_copy(acc.at[pl.ds(r0, 64)],
                        out_hbm.at[pl.ds(c * _NP + r0, 64)])
        return carry
    lax.fori_loop(0, _RPT // 64, cp, 0)


@functools.lru_cache(maxsize=None)
def _make_sc_agg(width):
    @functools.partial(
        pl.kernel,
        mesh=plsc.VectorSubcoreMesh(core_axis_name="c", subcore_axis_name="s"),
        out_type=jax.ShapeDtypeStruct((2 * _NP, width), _f32),
        compiler_params=pltpu.CompilerParams(needs_layout_passes=False,
                                             use_tc_tiling_on_sc=False),
        scratch_types=[
            pltpu.VMEM((_NP,), _f32),
            pltpu.VMEM((_NP,), _f32),
            pltpu.VMEM((2, _CH), jnp.int32),
            pltpu.VMEM((2, _CH), jnp.int32),
            pltpu.VMEM((_CH,), jnp.int32),
            pltpu.VMEM((_CH,), jnp.int32),
            pltpu.VMEM((_CH, width), _f32),
            pltpu.VMEM((_CH, width), _f32),
            pltpu.VMEM((_CH,), _f32),
            pltpu.VMEM((64, width), _f32),
            pltpu.VMEM_SHARED((_NP, width), _f32),
            pltpu.SemaphoreType.DMA,
            pltpu.SemaphoreType.DMA,
            pltpu.SemaphoreType.DMA,
            pltpu.SemaphoreType.DMA,
            pltpu.SemaphoreType.DMA,
            pltpu.SemaphoreType.DMA,
        ],
    )
    def _sc_agg(ebuf_hbm, as_hbm, ad_hbm, hp_hbm, out_hbm, *scratch):
        _sc_body(width, ebuf_hbm, as_hbm, ad_hbm, hp_hbm, out_hbm, *scratch)

    return _sc_agg


def kernel(x, edge_index, W1, att_src1, att_dst1, b1, W2, att_src2, att_dst2,
           b2):
    xp = jnp.zeros((_NP, _DIN), _f32).at[:_N].set(x)
    src = jnp.full((_EP,), 0, jnp.int32).at[:_E].set(
        edge_index[0].astype(jnp.int32))
    dst = (_JUNK + jnp.arange(_EP, dtype=jnp.int32) % 224).at[:_E].set(
        edge_index[1].astype(jnp.int32))
    ebuf = jnp.stack([src.reshape(_NTILES * _CPT, _CH),
                      dst.reshape(_NTILES * _CPT, _CH)], axis=1)

    asr1 = att_src1.reshape(1, _HID)
    adr1 = att_dst1.reshape(1, _HID)
    asr2 = att_src2.reshape(1, _NCLS)
    adr2 = att_dst2.reshape(1, _NCLS)

    hp1, als1, ald1, ws1 = _tc1(xp, W1, asr1, adr1)
    acc1 = _make_sc_agg(_W1)(ebuf, als1.reshape(_NP), ald1.reshape(_NP), hp1)
    hp2, als2, ald2, ws2 = _tc2(acc1[:_NP], acc1[_NP:], hp1, ws1,
                                b1.reshape(1, _HID), W2, asr2, adr2)
    acc2 = _make_sc_agg(_W2)(ebuf, als2.reshape(_NP), ald2.reshape(_NP), hp2)
    out = _tc3(acc2[:_NP], acc2[_NP:], hp2, ws2, b2.reshape(1, _NCLS))
    return out[:_N]


# speedup vs baseline: 30.0551x; 1.0104x over previous
import functools

import jax
import jax.numpy as jnp
from jax import lax
from jax.experimental import pallas as pl
from jax.experimental.pallas import tpu as pltpu
from jax.experimental.pallas import tpu_sc as plsc

_N = 10000
_NP = 10240
_DIN = 128
_HID = 32
_NCLS = 40
_W1 = 48
_ONE1 = 32
_W2 = 48
_ONE2 = 40
_E = 320000
_CH = 128
_NTILES = 32
_CPT = 80
_EPT = _CPT * _CH
_EP = _NTILES * _EPT
_JUNK = 10016
_RPT = _NP // 16
_BLK = 512
_GRID = _NP // _BLK

_f32 = jnp.float32


def _lrelu_exp(a):
    return jnp.exp(jnp.maximum(a, 0.2 * a))


def _tc1_body(x_ref, w_ref, asr_ref, adr_ref, hp_ref, als_ref, ald_ref, ws_ref):
    h = jnp.dot(x_ref[...], w_ref[...], preferred_element_type=_f32)
    als = jnp.sum(h * asr_ref[...], axis=1, keepdims=True)
    ald = jnp.sum(h * adr_ref[...], axis=1, keepdims=True)
    b = h.shape[0]
    hp_ref[...] = jnp.concatenate(
        [h, jnp.ones((b, 1), _f32), jnp.zeros((b, _W1 - _ONE1 - 1), _f32)],
        axis=1)
    als_ref[...] = als
    ald_ref[...] = ald
    ws_ref[...] = _lrelu_exp(als + ald)


def _tc2_body(a0_ref, a1_ref, hp1_ref, ws1_ref, b1_ref, w2_ref, asr_ref,
              adr_ref, hp_ref, als_ref, ald_ref, ws_ref):
    a0 = a0_ref[...]
    a1 = a1_ref[...]
    hp1 = hp1_ref[...]
    ws1 = ws1_ref[...]
    num = a0[:, :_HID] + a1[:, :_HID] + ws1 * hp1[:, :_HID]
    den = a0[:, _ONE1:_ONE1 + 1] + a1[:, _ONE1:_ONE1 + 1] + ws1 + 1e-16
    x2 = jax.nn.relu(num / den + b1_ref[...])
    h = jnp.dot(x2, w2_ref[...], preferred_element_type=_f32)
    als = jnp.sum(h * asr_ref[...], axis=1, keepdims=True)
    ald = jnp.sum(h * adr_ref[...], axis=1, keepdims=True)
    b = h.shape[0]
    hp_ref[...] = jnp.concatenate(
        [h, jnp.ones((b, 1), _f32), jnp.zeros((b, _W2 - _ONE2 - 1), _f32)],
        axis=1)
    als_ref[...] = als
    ald_ref[...] = ald
    ws_ref[...] = _lrelu_exp(als + ald)


def _tc3_body(a0_ref, a1_ref, hp2_ref, ws2_ref, b2_ref, out_ref):
    a0 = a0_ref[...]
    a1 = a1_ref[...]
    hp2 = hp2_ref[...]
    ws2 = ws2_ref[...]
    num = a0[:, :_NCLS] + a1[:, :_NCLS] + ws2 * hp2[:, :_NCLS]
    den = a0[:, _ONE2:_ONE2 + 1] + a1[:, _ONE2:_ONE2 + 1] + ws2 + 1e-16
    o = jax.nn.relu(num / den + b2_ref[...])
    m = jnp.max(o, axis=1, keepdims=True)
    e = o - m
    out_ref[...] = e - jnp.log(jnp.sum(jnp.exp(e), axis=1, keepdims=True))


def _row_block(width):
    return pl.BlockSpec((_BLK, width), lambda i: (i, 0))


def _full_block(shape):
    return pl.BlockSpec(shape, lambda i: (0,) * len(shape))


def _tc1(x, w1, asr, adr):
    return pl.pallas_call(
        _tc1_body,
        grid=(_GRID,),
        in_specs=[_row_block(_DIN), _full_block((_DIN, _HID)),
                  _full_block((1, _HID)), _full_block((1, _HID))],
        out_specs=[_row_block(_W1), _row_block(1), _row_block(1),
                   _row_block(1)],
        out_shape=[jax.ShapeDtypeStruct((_NP, _W1), _f32)] +
                  [jax.ShapeDtypeStruct((_NP, 1), _f32)] * 3,
    )(x, w1, asr, adr)


def _tc2(a0, a1, hp1, ws1, b1, w2, asr, adr):
    return pl.pallas_call(
        _tc2_body,
        grid=(_GRID,),
        in_specs=[_row_block(_W1), _row_block(_W1), _row_block(_W1),
                  _row_block(1), _full_block((1, _HID)),
                  _full_block((_HID, _NCLS)), _full_block((1, _NCLS)),
                  _full_block((1, _NCLS))],
        out_specs=[_row_block(_W2), _row_block(1), _row_block(1),
                   _row_block(1)],
        out_shape=[jax.ShapeDtypeStruct((_NP, _W2), _f32)] +
                  [jax.ShapeDtypeStruct((_NP, 1), _f32)] * 3,
    )(a0, a1, hp1, ws1, b1, w2, asr, adr)


def _tc3(a0, a1, hp2, ws2, b2):
    return pl.pallas_call(
        _tc3_body,
        grid=(_GRID,),
        in_specs=[_row_block(_W2), _row_block(_W2), _row_block(_W2),
                  _row_block(1), _full_block((1, _NCLS))],
        out_specs=_row_block(_NCLS),
        out_shape=jax.ShapeDtypeStruct((_NP, _NCLS), _f32),
    )(a0, a1, hp2, ws2, b2)


def _sc_body(width, ebuf_hbm, as_hbm, ad_hbm, hp_hbm, out_hbm,
             as_v, ad_v, eb0, eb1, db0, db1, rows0, rows1, wbuf, zbuf, acc,
             semi0, semi1, semg0, semg1, sems0, sems1):
    c = lax.axis_index("c")
    s = lax.axis_index("s")
    wid = s * 2 + c
    cbase = wid * _CPT
    eb = (eb0, eb1)
    db = (db0, db1)
    rows = (rows0, rows1)
    semi = (semi0, semi1)
    semg = (semg0, semg1)
    sems = (sems0, sems1)

    pltpu.sync_copy(as_hbm, as_v)
    pltpu.sync_copy(ad_hbm, ad_v)

    zoffs = [j * 16 for j in range(width // 16)]
    if width % 16:
        zoffs.append(width - 16)

    def zb(i, carry):
        for o in zoffs:
            zbuf[i, pl.ds(o, 16)] = jnp.zeros((16,), _f32)
        return carry
    lax.fori_loop(0, 64, zb, 0)

    def za(i, carry):
        pltpu.sync_copy(zbuf, acc.at[pl.ds(s * _RPT + i * 64, 64)])
        return carry
    lax.fori_loop(0, _RPT // 64, za, 0)
    plsc.subcore_barrier()

    pltpu.async_copy(ebuf_hbm.at[cbase], eb0, semi0)
    pltpu.async_copy(ebuf_hbm.at[cbase + 1], eb1, semi1)
    pltpu.make_async_copy(ebuf_hbm.at[cbase], eb0, semi0).wait()
    pltpu.async_copy(hp_hbm.at[eb0.at[0]], rows0, semg0)

    def pair(g2, carry):
        for b in range(2):
            g = g2 * 2 + b
            nb = 1 - b
            pltpu.make_async_copy(hp_hbm.at[eb[b].at[0]], rows[b],
                                  semg[b]).wait()
            for kk in range(_CH // 16):
                si = eb[b][0, pl.ds(kk * 16, 16)]
                di = eb[b][1, pl.ds(kk * 16, 16)]
                db[b][pl.ds(kk * 16, 16)] = di
                a = (plsc.load_gather(as_v, [si]) +
                     plsc.load_gather(ad_v, [di]))
                wbuf[pl.ds(kk * 16, 16)] = jnp.exp(jnp.maximum(a, 0.2 * a))
            nfull = width // 16
            tail = width % 16
            tail_mask = lax.iota(jnp.int32, 16) < (16 - tail)

            def scale(k, cy):
                wv = plsc.load_gather(wbuf, [jnp.broadcast_to(k, (16,))])
                for j in range(nfull):
                    rows[b][k, pl.ds(j * 16, 16)] = (
                        rows[b][k, pl.ds(j * 16, 16)] * wv)
                if tail:
                    wt = jnp.where(tail_mask, jnp.ones((16,), _f32), wv)
                    rows[b][k, pl.ds(width - 16, 16)] = (
                        rows[b][k, pl.ds(width - 16, 16)] * wt)
                return cy
            lax.fori_loop(0, _CH, scale, 0, unroll=4)
            pltpu.async_copy(rows[b], acc.at[db[b]], sems[b], add=True)
            @pl.when(g + 2 < _CPT)
            def _():
                pltpu.async_copy(ebuf_hbm.at[cbase + g + 2], eb[b], semi[b])
            @pl.when(g >= 1)
            def _():
                pltpu.make_async_copy(rows[nb], acc.at[db[nb]],
                                      sems[nb]).wait()
            @pl.when(g + 1 < _CPT)
            def _():
                pltpu.make_async_copy(ebuf_hbm.at[cbase + g + 1], eb[nb],
                                      semi[nb]).wait()
                pltpu.async_copy(hp_hbm.at[eb[nb].at[0]], rows[nb], semg[nb])
        return carry
    lax.fori_loop(0, _CPT // 2, pair, 0)
    pltpu.make_async_copy(rows[(_CPT - 1) % 2], acc.at[db[(_CPT - 1) % 2]],
                          sems[(_CPT - 1) % 2]).wait()
    plsc.subcore_barrier()

    def cp(i, carry):
        r0 = s * _RPT + i * 64
        pltpu.sync_copy(acc.at[pl.ds(r0, 64)],
                        out_hbm.at[pl.ds(c * _NP + r0, 64)])
        return carry
    lax.fori_loop(0, _RPT // 64, cp, 0)


@functools.lru_cache(maxsize=None)
def _make_sc_agg(width):
    @functools.partial(
        pl.kernel,
        mesh=plsc.VectorSubcoreMesh(core_axis_name="c", subcore_axis_name="s"),
        out_type=jax.ShapeDtypeStruct((2 * _NP, width), _f32),
        compiler_params=pltpu.CompilerParams(needs_layout_passes=False,
                                             use_tc_tiling_on_sc=False),
        scratch_types=[
            pltpu.VMEM((_NP,), _f32),
            pltpu.VMEM((_NP,), _f32),
            pltpu.VMEM((2, _CH), jnp.int32),
            pltpu.VMEM((2, _CH), jnp.int32),
            pltpu.VMEM((_CH,), jnp.int32),
            pltpu.VMEM((_CH,), jnp.int32),
            pltpu.VMEM((_CH, width), _f32),
            pltpu.VMEM((_CH, width), _f32),
            pltpu.VMEM((_CH,), _f32),
            pltpu.VMEM((64, width), _f32),
            pltpu.VMEM_SHARED((_NP, width), _f32),
            pltpu.SemaphoreType.DMA,
            pltpu.SemaphoreType.DMA,
            pltpu.SemaphoreType.DMA,
            pltpu.SemaphoreType.DMA,
            pltpu.SemaphoreType.DMA,
            pltpu.SemaphoreType.DMA,
        ],
    )
    def _sc_agg(ebuf_hbm, as_hbm, ad_hbm, hp_hbm, out_hbm, *scratch):
        _sc_body(width, ebuf_hbm, as_hbm, ad_hbm, hp_hbm, out_hbm, *scratch)

    return _sc_agg


def kernel(x, edge_index, W1, att_src1, att_dst1, b1, W2, att_src2, att_dst2,
           b2):
    xp = jnp.zeros((_NP, _DIN), _f32).at[:_N].set(x)
    src = jnp.full((_EP,), 0, jnp.int32).at[:_E].set(
        edge_index[0].astype(jnp.int32))
    dst = (_JUNK + jnp.arange(_EP, dtype=jnp.int32) % 224).at[:_E].set(
        edge_index[1].astype(jnp.int32))
    ebuf = jnp.stack([src.reshape(_NTILES * _CPT, _CH),
                      dst.reshape(_NTILES * _CPT, _CH)], axis=1)
    ebuf = (ebuf.reshape(_CPT, _NTILES, 2, _CH)
            .transpose(1, 0, 2, 3).reshape(_NTILES * _CPT, 2, _CH))

    asr1 = att_src1.reshape(1, _HID)
    adr1 = att_dst1.reshape(1, _HID)
    asr2 = att_src2.reshape(1, _NCLS)
    adr2 = att_dst2.reshape(1, _NCLS)

    hp1, als1, ald1, ws1 = _tc1(xp, W1, asr1, adr1)
    acc1 = _make_sc_agg(_W1)(ebuf, als1.reshape(_NP), ald1.reshape(_NP), hp1)
    hp2, als2, ald2, ws2 = _tc2(acc1[:_NP], acc1[_NP:], hp1, ws1,
                                b1.reshape(1, _HID), W2, asr2, adr2)
    acc2 = _make_sc_agg(_W2)(ebuf, als2.reshape(_NP), ald2.reshape(_NP), hp2)
    out = _tc3(acc2[:_NP], acc2[_NP:], hp2, ws2, b2.reshape(1, _NCLS))
    return out[:_N]


# speedup vs baseline: 33.1793x; 1.1040x over previous
import functools

import jax
import jax.numpy as jnp
from jax import lax
from jax.experimental import pallas as pl
from jax.experimental.pallas import tpu as pltpu
from jax.experimental.pallas import tpu_sc as plsc

_N = 10000
_NP = 10240
_DIN = 128
_HID = 32
_NCLS = 40
_W1 = 48
_ONE1 = 32
_W2 = 48
_ONE2 = 40
_E = 320000
_CH = 128
_NTILES = 32
_CPT = 80
_EPT = _CPT * _CH
_EP = _NTILES * _EPT
_JUNK = 10016
_RPT = _NP // 16
_BLK = 512
_GRID = _NP // _BLK

_f32 = jnp.float32


def _lrelu_exp(a):
    return jnp.exp(jnp.maximum(a, 0.2 * a))


def _tc1_body(x_ref, w_ref, asr_ref, adr_ref, hp_ref, als_ref, ald_ref, ws_ref):
    h = jnp.dot(x_ref[...], w_ref[...], preferred_element_type=_f32)
    als = jnp.sum(h * asr_ref[...], axis=1, keepdims=True)
    ald = jnp.sum(h * adr_ref[...], axis=1, keepdims=True)
    b = h.shape[0]
    hp_ref[...] = jnp.concatenate(
        [h, jnp.ones((b, 1), _f32), jnp.zeros((b, _W1 - _ONE1 - 1), _f32)],
        axis=1)
    als_ref[...] = als
    ald_ref[...] = ald
    ws_ref[...] = _lrelu_exp(als + ald)


def _tc2_body(a0_ref, a1_ref, hp1_ref, ws1_ref, b1_ref, w2_ref, asr_ref,
              adr_ref, hp_ref, als_ref, ald_ref, ws_ref):
    a0 = a0_ref[...]
    a1 = a1_ref[...]
    hp1 = hp1_ref[...]
    ws1 = ws1_ref[...]
    num = a0[:, :_HID] + a1[:, :_HID] + ws1 * hp1[:, :_HID]
    den = a0[:, _ONE1:_ONE1 + 1] + a1[:, _ONE1:_ONE1 + 1] + ws1 + 1e-16
    x2 = jax.nn.relu(num / den + b1_ref[...])
    h = jnp.dot(x2, w2_ref[...], preferred_element_type=_f32)
    als = jnp.sum(h * asr_ref[...], axis=1, keepdims=True)
    ald = jnp.sum(h * adr_ref[...], axis=1, keepdims=True)
    b = h.shape[0]
    hp_ref[...] = jnp.concatenate(
        [h, jnp.ones((b, 1), _f32), jnp.zeros((b, _W2 - _ONE2 - 1), _f32)],
        axis=1)
    als_ref[...] = als
    ald_ref[...] = ald
    ws_ref[...] = _lrelu_exp(als + ald)


def _tc3_body(a0_ref, a1_ref, hp2_ref, ws2_ref, b2_ref, out_ref):
    a0 = a0_ref[...]
    a1 = a1_ref[...]
    hp2 = hp2_ref[...]
    ws2 = ws2_ref[...]
    num = a0[:, :_NCLS] + a1[:, :_NCLS] + ws2 * hp2[:, :_NCLS]
    den = a0[:, _ONE2:_ONE2 + 1] + a1[:, _ONE2:_ONE2 + 1] + ws2 + 1e-16
    o = jax.nn.relu(num / den + b2_ref[...])
    m = jnp.max(o, axis=1, keepdims=True)
    e = o - m
    out_ref[...] = e - jnp.log(jnp.sum(jnp.exp(e), axis=1, keepdims=True))


def _row_block(width):
    return pl.BlockSpec((_BLK, width), lambda i: (i, 0))


def _full_block(shape):
    return pl.BlockSpec(shape, lambda i: (0,) * len(shape))


def _tc1(x, w1, asr, adr):
    return pl.pallas_call(
        _tc1_body,
        grid=(_GRID,),
        in_specs=[_row_block(_DIN), _full_block((_DIN, _HID)),
                  _full_block((1, _HID)), _full_block((1, _HID))],
        out_specs=[_row_block(_W1), _row_block(1), _row_block(1),
                   _row_block(1)],
        out_shape=[jax.ShapeDtypeStruct((_NP, _W1), _f32)] +
                  [jax.ShapeDtypeStruct((_NP, 1), _f32)] * 3,
    )(x, w1, asr, adr)


def _tc2(a0, a1, hp1, ws1, b1, w2, asr, adr):
    return pl.pallas_call(
        _tc2_body,
        grid=(_GRID,),
        in_specs=[_row_block(_W1), _row_block(_W1), _row_block(_W1),
                  _row_block(1), _full_block((1, _HID)),
                  _full_block((_HID, _NCLS)), _full_block((1, _NCLS)),
                  _full_block((1, _NCLS))],
        out_specs=[_row_block(_W2), _row_block(1), _row_block(1),
                   _row_block(1)],
        out_shape=[jax.ShapeDtypeStruct((_NP, _W2), _f32)] +
                  [jax.ShapeDtypeStruct((_NP, 1), _f32)] * 3,
    )(a0, a1, hp1, ws1, b1, w2, asr, adr)


def _tc3(a0, a1, hp2, ws2, b2):
    return pl.pallas_call(
        _tc3_body,
        grid=(_GRID,),
        in_specs=[_row_block(_W2), _row_block(_W2), _row_block(_W2),
                  _row_block(1), _full_block((1, _NCLS))],
        out_specs=_row_block(_NCLS),
        out_shape=jax.ShapeDtypeStruct((_NP, _NCLS), _f32),
    )(a0, a1, hp2, ws2, b2)


def _sc_body(width, ebuf_hbm, as_hbm, ad_hbm, hp_hbm, out_hbm,
             as_v, ad_v, eb0, eb1, db0, db1, rows0, rows1, wbuf, zbuf, acc,
             semi0, semi1, semg0, semg1, sems0, sems1):
    c = lax.axis_index("c")
    s = lax.axis_index("s")
    wid = s * 2 + c
    cbase = wid * _CPT
    eb = (eb0, eb1)
    db = (db0, db1)
    rows = (rows0, rows1)
    semi = (semi0, semi1)
    semg = (semg0, semg1)
    sems = (sems0, sems1)

    pltpu.sync_copy(as_hbm, as_v)
    pltpu.sync_copy(ad_hbm, ad_v)

    zoffs = [j * 16 for j in range(width // 16)]
    if width % 16:
        zoffs.append(width - 16)

    def zb(i, carry):
        for o in zoffs:
            zbuf[i, pl.ds(o, 16)] = jnp.zeros((16,), _f32)
        return carry
    lax.fori_loop(0, 64, zb, 0)

    def za(i, carry):
        pltpu.sync_copy(zbuf, acc.at[pl.ds(s * _RPT + i * 64, 64)])
        return carry
    lax.fori_loop(0, _RPT // 64, za, 0)
    plsc.subcore_barrier()

    pltpu.async_copy(ebuf_hbm.at[cbase], eb0, semi0)
    pltpu.async_copy(ebuf_hbm.at[cbase + 1], eb1, semi1)
    pltpu.make_async_copy(ebuf_hbm.at[cbase], eb0, semi0).wait()
    pltpu.async_copy(hp_hbm.at[eb0.at[0]], rows0, semg0)

    def pair(g2, carry):
        for b in range(2):
            g = g2 * 2 + b
            nb = 1 - b
            pltpu.make_async_copy(hp_hbm.at[eb[b].at[0]], rows[b],
                                  semg[b]).wait()
            @pl.when(g >= 1)
            def _():
                pltpu.make_async_copy(rows[nb], acc.at[db[nb]],
                                      sems[nb]).wait()
            @pl.when(g + 1 < _CPT)
            def _():
                pltpu.make_async_copy(ebuf_hbm.at[cbase + g + 1], eb[nb],
                                      semi[nb]).wait()
                pltpu.async_copy(hp_hbm.at[eb[nb].at[0]], rows[nb], semg[nb])
            for kk in range(_CH // 16):
                si = eb[b][0, pl.ds(kk * 16, 16)]
                di = eb[b][1, pl.ds(kk * 16, 16)]
                db[b][pl.ds(kk * 16, 16)] = di
                a = (plsc.load_gather(as_v, [si]) +
                     plsc.load_gather(ad_v, [di]))
                wbuf[pl.ds(kk * 16, 16)] = jnp.exp(jnp.maximum(a, 0.2 * a))
            nfull = width // 16
            tail = width % 16
            tail_mask = lax.iota(jnp.int32, 16) < (16 - tail)

            def scale(k, cy):
                wv = plsc.load_gather(wbuf, [jnp.broadcast_to(k, (16,))])
                for j in range(nfull):
                    rows[b][k, pl.ds(j * 16, 16)] = (
                        rows[b][k, pl.ds(j * 16, 16)] * wv)
                if tail:
                    wt = jnp.where(tail_mask, jnp.ones((16,), _f32), wv)
                    rows[b][k, pl.ds(width - 16, 16)] = (
                        rows[b][k, pl.ds(width - 16, 16)] * wt)
                return cy
            lax.fori_loop(0, _CH, scale, 0, unroll=4)
            pltpu.async_copy(rows[b], acc.at[db[b]], sems[b], add=True)
            @pl.when(g + 2 < _CPT)
            def _():
                pltpu.async_copy(ebuf_hbm.at[cbase + g + 2], eb[b], semi[b])
        return carry
    lax.fori_loop(0, _CPT // 2, pair, 0)
    pltpu.make_async_copy(rows[(_CPT - 1) % 2], acc.at[db[(_CPT - 1) % 2]],
                          sems[(_CPT - 1) % 2]).wait()
    plsc.subcore_barrier()

    def cp(i, carry):
        r0 = s * _RPT + i * 64
        pltpu.sync_copy(acc.at[pl.ds(r0, 64)],
                        out_hbm.at[pl.ds(c * _NP + r0, 64)])
        return carry
    lax.fori_loop(0, _RPT // 64, cp, 0)


@functools.lru_cache(maxsize=None)
def _make_sc_agg(width):
    @functools.partial(
        pl.kernel,
        mesh=plsc.VectorSubcoreMesh(core_axis_name="c", subcore_axis_name="s"),
        out_type=jax.ShapeDtypeStruct((2 * _NP, width), _f32),
        compiler_params=pltpu.CompilerParams(needs_layout_passes=False,
                                             use_tc_tiling_on_sc=False),
        scratch_types=[
            pltpu.VMEM((_NP,), _f32),
            pltpu.VMEM((_NP,), _f32),
            pltpu.VMEM((2, _CH), jnp.int32),
            pltpu.VMEM((2, _CH), jnp.int32),
            pltpu.VMEM((_CH,), jnp.int32),
            pltpu.VMEM((_CH,), jnp.int32),
            pltpu.VMEM((_CH, width), _f32),
            pltpu.VMEM((_CH, width), _f32),
            pltpu.VMEM((_CH,), _f32),
            pltpu.VMEM((64, width), _f32),
            pltpu.VMEM_SHARED((_NP, width), _f32),
            pltpu.SemaphoreType.DMA,
            pltpu.SemaphoreType.DMA,
            pltpu.SemaphoreType.DMA,
            pltpu.SemaphoreType.DMA,
            pltpu.SemaphoreType.DMA,
            pltpu.SemaphoreType.DMA,
        ],
    )
    def _sc_agg(ebuf_hbm, as_hbm, ad_hbm, hp_hbm, out_hbm, *scratch):
        _sc_body(width, ebuf_hbm, as_hbm, ad_hbm, hp_hbm, out_hbm, *scratch)

    return _sc_agg


def kernel(x, edge_index, W1, att_src1, att_dst1, b1, W2, att_src2, att_dst2,
           b2):
    xp = jnp.zeros((_NP, _DIN), _f32).at[:_N].set(x)
    src = jnp.full((_EP,), 0, jnp.int32).at[:_E].set(
        edge_index[0].astype(jnp.int32))
    dst = (_JUNK + jnp.arange(_EP, dtype=jnp.int32) % 224).at[:_E].set(
        edge_index[1].astype(jnp.int32))
    ebuf = jnp.stack([src.reshape(_NTILES * _CPT, _CH),
                      dst.reshape(_NTILES * _CPT, _CH)], axis=1)
    ebuf = (ebuf.reshape(_CPT, _NTILES, 2, _CH)
            .transpose(1, 0, 2, 3).reshape(_NTILES * _CPT, 2, _CH))

    asr1 = att_src1.reshape(1, _HID)
    adr1 = att_dst1.reshape(1, _HID)
    asr2 = att_src2.reshape(1, _NCLS)
    adr2 = att_dst2.reshape(1, _NCLS)

    hp1, als1, ald1, ws1 = _tc1(xp, W1, asr1, adr1)
    acc1 = _make_sc_agg(_W1)(ebuf, als1.reshape(_NP), ald1.reshape(_NP), hp1)
    hp2, als2, ald2, ws2 = _tc2(acc1[:_NP], acc1[_NP:], hp1, ws1,
                                b1.reshape(1, _HID), W2, asr2, adr2)
    acc2 = _make_sc_agg(_W2)(ebuf, als2.reshape(_NP), ald2.reshape(_NP), hp2)
    out = _tc3(acc2[:_NP], acc2[_NP:], hp2, ws2, b2.reshape(1, _NCLS))
    return out[:_N]


# speedup vs baseline: 34.8676x; 1.0509x over previous
import functools

import jax
import jax.numpy as jnp
from jax import lax
from jax.experimental import pallas as pl
from jax.experimental.pallas import tpu as pltpu
from jax.experimental.pallas import tpu_sc as plsc

_N = 10000
_NP = 10240
_DIN = 128
_HID = 32
_NCLS = 40
_W1 = 48
_ONE1 = 32
_W2 = 48
_ONE2 = 40
_E = 320000
_CH = 128
_NTILES = 32
_SUB = 2
_CE = _SUB * _CH
_CPT = 40
_EPT = _CPT * _CE
_EP = _NTILES * _EPT
_JUNK = 10016
_RPT = _NP // 16
_BLK = 512
_GRID = _NP // _BLK

_f32 = jnp.float32


def _lrelu_exp(a):
    return jnp.exp(jnp.maximum(a, 0.2 * a))


def _tc1_body(x_ref, w_ref, asr_ref, adr_ref, hp_ref, als_ref, ald_ref, ws_ref):
    h = jnp.dot(x_ref[...], w_ref[...], preferred_element_type=_f32)
    als = jnp.sum(h * asr_ref[...], axis=1, keepdims=True)
    ald = jnp.sum(h * adr_ref[...], axis=1, keepdims=True)
    b = h.shape[0]
    hp_ref[...] = jnp.concatenate(
        [h, jnp.ones((b, 1), _f32), jnp.zeros((b, _W1 - _ONE1 - 1), _f32)],
        axis=1)
    als_ref[...] = als
    ald_ref[...] = ald
    ws_ref[...] = _lrelu_exp(als + ald)


def _tc2_body(a0_ref, a1_ref, hp1_ref, ws1_ref, b1_ref, w2_ref, asr_ref,
              adr_ref, hp_ref, als_ref, ald_ref, ws_ref):
    a0 = a0_ref[...]
    a1 = a1_ref[...]
    hp1 = hp1_ref[...]
    ws1 = ws1_ref[...]
    num = a0[:, :_HID] + a1[:, :_HID] + ws1 * hp1[:, :_HID]
    den = a0[:, _ONE1:_ONE1 + 1] + a1[:, _ONE1:_ONE1 + 1] + ws1 + 1e-16
    x2 = jax.nn.relu(num / den + b1_ref[...])
    h = jnp.dot(x2, w2_ref[...], preferred_element_type=_f32)
    als = jnp.sum(h * asr_ref[...], axis=1, keepdims=True)
    ald = jnp.sum(h * adr_ref[...], axis=1, keepdims=True)
    b = h.shape[0]
    hp_ref[...] = jnp.concatenate(
        [h, jnp.ones((b, 1), _f32), jnp.zeros((b, _W2 - _ONE2 - 1), _f32)],
        axis=1)
    als_ref[...] = als
    ald_ref[...] = ald
    ws_ref[...] = _lrelu_exp(als + ald)


def _tc3_body(a0_ref, a1_ref, hp2_ref, ws2_ref, b2_ref, out_ref):
    a0 = a0_ref[...]
    a1 = a1_ref[...]
    hp2 = hp2_ref[...]
    ws2 = ws2_ref[...]
    num = a0[:, :_NCLS] + a1[:, :_NCLS] + ws2 * hp2[:, :_NCLS]
    den = a0[:, _ONE2:_ONE2 + 1] + a1[:, _ONE2:_ONE2 + 1] + ws2 + 1e-16
    o = jax.nn.relu(num / den + b2_ref[...])
    m = jnp.max(o, axis=1, keepdims=True)
    e = o - m
    out_ref[...] = e - jnp.log(jnp.sum(jnp.exp(e), axis=1, keepdims=True))


def _row_block(width):
    return pl.BlockSpec((_BLK, width), lambda i: (i, 0))


def _full_block(shape):
    return pl.BlockSpec(shape, lambda i: (0,) * len(shape))


def _tc1(x, w1, asr, adr):
    return pl.pallas_call(
        _tc1_body,
        grid=(_GRID,),
        in_specs=[_row_block(_DIN), _full_block((_DIN, _HID)),
                  _full_block((1, _HID)), _full_block((1, _HID))],
        out_specs=[_row_block(_W1), _row_block(1), _row_block(1),
                   _row_block(1)],
        out_shape=[jax.ShapeDtypeStruct((_NP, _W1), _f32)] +
                  [jax.ShapeDtypeStruct((_NP, 1), _f32)] * 3,
    )(x, w1, asr, adr)


def _tc2(a0, a1, hp1, ws1, b1, w2, asr, adr):
    return pl.pallas_call(
        _tc2_body,
        grid=(_GRID,),
        in_specs=[_row_block(_W1), _row_block(_W1), _row_block(_W1),
                  _row_block(1), _full_block((1, _HID)),
                  _full_block((_HID, _NCLS)), _full_block((1, _NCLS)),
                  _full_block((1, _NCLS))],
        out_specs=[_row_block(_W2), _row_block(1), _row_block(1),
                   _row_block(1)],
        out_shape=[jax.ShapeDtypeStruct((_NP, _W2), _f32)] +
                  [jax.ShapeDtypeStruct((_NP, 1), _f32)] * 3,
    )(a0, a1, hp1, ws1, b1, w2, asr, adr)


def _tc3(a0, a1, hp2, ws2, b2):
    return pl.pallas_call(
        _tc3_body,
        grid=(_GRID,),
        in_specs=[_row_block(_W2), _row_block(_W2), _row_block(_W2),
                  _row_block(1), _full_block((1, _NCLS))],
        out_specs=_row_block(_NCLS),
        out_shape=jax.ShapeDtypeStruct((_NP, _NCLS), _f32),
    )(a0, a1, hp2, ws2, b2)


def _sc_body(width, ebuf_hbm, as_hbm, ad_hbm, hp_hbm, out_hbm,
             as_v, ad_v, eb0, eb1, db0, db1, rows0, rows1, wbuf, zbuf, acc,
             semi0, semi1, semg0, semg1, sems0, sems1):
    c = lax.axis_index("c")
    s = lax.axis_index("s")
    wid = s * 2 + c
    cbase = wid * _CPT
    eb = (eb0, eb1)
    db = (db0, db1)
    rows = (rows0, rows1)
    semi = (semi0, semi1)
    semg = (semg0, semg1)
    sems = (sems0, sems1)

    def gather_start(i, r):
        for h in range(_SUB):
            pltpu.async_copy(hp_hbm.at[eb[i].at[h]],
                             rows[i].at[pl.ds(h * _CH, _CH)], semg[i])

    def gather_wait(i):
        for h in range(_SUB):
            pltpu.make_async_copy(hp_hbm.at[eb[i].at[h]],
                                  rows[i].at[pl.ds(h * _CH, _CH)],
                                  semg[i]).wait()

    def scatter_start(i):
        for h in range(_SUB):
            pltpu.async_copy(rows[i].at[pl.ds(h * _CH, _CH)],
                             acc.at[db[i].at[h]], sems[i], add=True)

    def scatter_wait(i):
        for h in range(_SUB):
            pltpu.make_async_copy(rows[i].at[pl.ds(h * _CH, _CH)],
                                  acc.at[db[i].at[h]], sems[i]).wait()

    pltpu.sync_copy(as_hbm, as_v)
    pltpu.sync_copy(ad_hbm, ad_v)

    zoffs = [j * 16 for j in range(width // 16)]
    if width % 16:
        zoffs.append(width - 16)

    def zb(i, carry):
        for o in zoffs:
            zbuf[i, pl.ds(o, 16)] = jnp.zeros((16,), _f32)
        return carry
    lax.fori_loop(0, 64, zb, 0)

    def za(i, carry):
        pltpu.sync_copy(zbuf, acc.at[pl.ds(s * _RPT + i * 64, 64)])
        return carry
    lax.fori_loop(0, _RPT // 64, za, 0)
    plsc.subcore_barrier()

    pltpu.async_copy(ebuf_hbm.at[cbase], eb0, semi0)
    pltpu.async_copy(ebuf_hbm.at[cbase + 1], eb1, semi1)
    pltpu.make_async_copy(ebuf_hbm.at[cbase], eb0, semi0).wait()
    gather_start(0, 0)

    def pair(g2, carry):
        for b in range(2):
            g = g2 * 2 + b
            nb = 1 - b
            gather_wait(b)
            @pl.when(g >= 1)
            def _():
                scatter_wait(nb)
            @pl.when(g + 1 < _CPT)
            def _():
                pltpu.make_async_copy(ebuf_hbm.at[cbase + g + 1], eb[nb],
                                      semi[nb]).wait()
                gather_start(nb, 0)
            for kk in range(_CE // 16):
                h, o = kk // (_CH // 16), (kk % (_CH // 16)) * 16
                si = eb[b][h, pl.ds(o, 16)]
                di = eb[b][_SUB + h, pl.ds(o, 16)]
                db[b][h, pl.ds(o, 16)] = di
                a = (plsc.load_gather(as_v, [si]) +
                     plsc.load_gather(ad_v, [di]))
                wbuf[pl.ds(kk * 16, 16)] = jnp.exp(jnp.maximum(a, 0.2 * a))
            nfull = width // 16
            tail = width % 16
            tail_mask = lax.iota(jnp.int32, 16) < (16 - tail)

            def scale(k, cy):
                wv = plsc.load_gather(wbuf, [jnp.broadcast_to(k, (16,))])
                for j in range(nfull):
                    rows[b][k, pl.ds(j * 16, 16)] = (
                        rows[b][k, pl.ds(j * 16, 16)] * wv)
                if tail:
                    wt = jnp.where(tail_mask, jnp.ones((16,), _f32), wv)
                    rows[b][k, pl.ds(width - 16, 16)] = (
                        rows[b][k, pl.ds(width - 16, 16)] * wt)
                return cy
            lax.fori_loop(0, _CE, scale, 0, unroll=4)
            scatter_start(b)
            @pl.when(g + 2 < _CPT)
            def _():
                pltpu.async_copy(ebuf_hbm.at[cbase + g + 2], eb[b], semi[b])
        return carry
    lax.fori_loop(0, _CPT // 2, pair, 0)
    scatter_wait((_CPT - 1) % 2)
    plsc.subcore_barrier()

    def cp(i, carry):
        r0 = s * _RPT + i * 64
        pltpu.sync_copy(acc.at[pl.ds(r0, 64)],
                        out_hbm.at[pl.ds(c * _NP + r0, 64)])
        return carry
    lax.fori_loop(0, _RPT // 64, cp, 0)


@functools.lru_cache(maxsize=None)
def _make_sc_agg(width):
    @functools.partial(
        pl.kernel,
        mesh=plsc.VectorSubcoreMesh(core_axis_name="c", subcore_axis_name="s"),
        out_type=jax.ShapeDtypeStruct((2 * _NP, width), _f32),
        compiler_params=pltpu.CompilerParams(needs_layout_passes=False,
                                             use_tc_tiling_on_sc=False),
        scratch_types=[
            pltpu.VMEM((_NP,), _f32),
            pltpu.VMEM((_NP,), _f32),
            pltpu.VMEM((2 * _SUB, _CH), jnp.int32),
            pltpu.VMEM((2 * _SUB, _CH), jnp.int32),
            pltpu.VMEM((_SUB, _CH), jnp.int32),
            pltpu.VMEM((_SUB, _CH), jnp.int32),
            pltpu.VMEM((_CE, width), _f32),
            pltpu.VMEM((_CE, width), _f32),
            pltpu.VMEM((_CE,), _f32),
            pltpu.VMEM((64, width), _f32),
            pltpu.VMEM_SHARED((_NP, width), _f32),
            pltpu.SemaphoreType.DMA,
            pltpu.SemaphoreType.DMA,
            pltpu.SemaphoreType.DMA,
            pltpu.SemaphoreType.DMA,
            pltpu.SemaphoreType.DMA,
            pltpu.SemaphoreType.DMA,
        ],
    )
    def _sc_agg(ebuf_hbm, as_hbm, ad_hbm, hp_hbm, out_hbm, *scratch):
        _sc_body(width, ebuf_hbm, as_hbm, ad_hbm, hp_hbm, out_hbm, *scratch)

    return _sc_agg


def kernel(x, edge_index, W1, att_src1, att_dst1, b1, W2, att_src2, att_dst2,
           b2):
    xp = jnp.zeros((_NP, _DIN), _f32).at[:_N].set(x)
    src = jnp.full((_EP,), 0, jnp.int32).at[:_E].set(
        edge_index[0].astype(jnp.int32))
    dst = (_JUNK + jnp.arange(_EP, dtype=jnp.int32) % 224).at[:_E].set(
        edge_index[1].astype(jnp.int32))
    ebuf = jnp.concatenate(
        [src.reshape(_NTILES * _CPT, _SUB, _CH),
         dst.reshape(_NTILES * _CPT, _SUB, _CH)], axis=1)
    ebuf = (ebuf.reshape(_CPT, _NTILES, 2 * _SUB, _CH)
            .transpose(1, 0, 2, 3).reshape(_NTILES * _CPT, 2 * _SUB, _CH))

    asr1 = att_src1.reshape(1, _HID)
    adr1 = att_dst1.reshape(1, _HID)
    asr2 = att_src2.reshape(1, _NCLS)
    adr2 = att_dst2.reshape(1, _NCLS)

    hp1, als1, ald1, ws1 = _tc1(xp, W1, asr1, adr1)
    acc1 = _make_sc_agg(_W1)(ebuf, als1.reshape(_NP), ald1.reshape(_NP), hp1)
    hp2, als2, ald2, ws2 = _tc2(acc1[:_NP], acc1[_NP:], hp1, ws1,
                                b1.reshape(1, _HID), W2, asr2, adr2)
    acc2 = _make_sc_agg(_W2)(ebuf, als2.reshape(_NP), ald2.reshape(_NP), hp2)
    out = _tc3(acc2[:_NP], acc2[_NP:], hp2, ws2, b2.reshape(1, _NCLS))
    return out[:_N]


# speedup vs baseline: 35.0865x; 1.0063x over previous
import functools

import jax
import jax.numpy as jnp
from jax import lax
from jax.experimental import pallas as pl
from jax.experimental.pallas import tpu as pltpu
from jax.experimental.pallas import tpu_sc as plsc

_N = 10000
_NP = 10240
_DIN = 128
_HID = 32
_NCLS = 40
_W1 = 48
_ONE1 = 32
_W2 = 48
_ONE2 = 40
_E = 320000
_CH = 128
_NTILES = 32
_SUB = 4
_CE = _SUB * _CH
_CPT = 20
_EPT = _CPT * _CE
_EP = _NTILES * _EPT
_JUNK = 10016
_RPT = _NP // 16
_BLK = 512
_GRID = _NP // _BLK

_f32 = jnp.float32


def _lrelu_exp(a):
    return jnp.exp(jnp.maximum(a, 0.2 * a))


def _tc1_body(x_ref, w_ref, asr_ref, adr_ref, hp_ref, als_ref, ald_ref, ws_ref):
    h = jnp.dot(x_ref[...], w_ref[...], preferred_element_type=_f32)
    als = jnp.sum(h * asr_ref[...], axis=1, keepdims=True)
    ald = jnp.sum(h * adr_ref[...], axis=1, keepdims=True)
    b = h.shape[0]
    hp_ref[...] = jnp.concatenate(
        [h, jnp.ones((b, 1), _f32), jnp.zeros((b, _W1 - _ONE1 - 1), _f32)],
        axis=1)
    als_ref[...] = als
    ald_ref[...] = ald
    ws_ref[...] = _lrelu_exp(als + ald)


def _tc2_body(a0_ref, a1_ref, hp1_ref, ws1_ref, b1_ref, w2_ref, asr_ref,
              adr_ref, hp_ref, als_ref, ald_ref, ws_ref):
    a0 = a0_ref[...]
    a1 = a1_ref[...]
    hp1 = hp1_ref[...]
    ws1 = ws1_ref[...]
    num = a0[:, :_HID] + a1[:, :_HID] + ws1 * hp1[:, :_HID]
    den = a0[:, _ONE1:_ONE1 + 1] + a1[:, _ONE1:_ONE1 + 1] + ws1 + 1e-16
    x2 = jax.nn.relu(num / den + b1_ref[...])
    h = jnp.dot(x2, w2_ref[...], preferred_element_type=_f32)
    als = jnp.sum(h * asr_ref[...], axis=1, keepdims=True)
    ald = jnp.sum(h * adr_ref[...], axis=1, keepdims=True)
    b = h.shape[0]
    hp_ref[...] = jnp.concatenate(
        [h, jnp.ones((b, 1), _f32), jnp.zeros((b, _W2 - _ONE2 - 1), _f32)],
        axis=1)
    als_ref[...] = als
    ald_ref[...] = ald
    ws_ref[...] = _lrelu_exp(als + ald)


def _tc3_body(a0_ref, a1_ref, hp2_ref, ws2_ref, b2_ref, out_ref):
    a0 = a0_ref[...]
    a1 = a1_ref[...]
    hp2 = hp2_ref[...]
    ws2 = ws2_ref[...]
    num = a0[:, :_NCLS] + a1[:, :_NCLS] + ws2 * hp2[:, :_NCLS]
    den = a0[:, _ONE2:_ONE2 + 1] + a1[:, _ONE2:_ONE2 + 1] + ws2 + 1e-16
    o = jax.nn.relu(num / den + b2_ref[...])
    m = jnp.max(o, axis=1, keepdims=True)
    e = o - m
    out_ref[...] = e - jnp.log(jnp.sum(jnp.exp(e), axis=1, keepdims=True))


def _row_block(width):
    return pl.BlockSpec((_BLK, width), lambda i: (i, 0))


def _full_block(shape):
    return pl.BlockSpec(shape, lambda i: (0,) * len(shape))


def _tc1(x, w1, asr, adr):
    return pl.pallas_call(
        _tc1_body,
        grid=(_GRID,),
        in_specs=[_row_block(_DIN), _full_block((_DIN, _HID)),
                  _full_block((1, _HID)), _full_block((1, _HID))],
        out_specs=[_row_block(_W1), _row_block(1), _row_block(1),
                   _row_block(1)],
        out_shape=[jax.ShapeDtypeStruct((_NP, _W1), _f32)] +
                  [jax.ShapeDtypeStruct((_NP, 1), _f32)] * 3,
    )(x, w1, asr, adr)


def _tc2(a0, a1, hp1, ws1, b1, w2, asr, adr):
    return pl.pallas_call(
        _tc2_body,
        grid=(_GRID,),
        in_specs=[_row_block(_W1), _row_block(_W1), _row_block(_W1),
                  _row_block(1), _full_block((1, _HID)),
                  _full_block((_HID, _NCLS)), _full_block((1, _NCLS)),
                  _full_block((1, _NCLS))],
        out_specs=[_row_block(_W2), _row_block(1), _row_block(1),
                   _row_block(1)],
        out_shape=[jax.ShapeDtypeStruct((_NP, _W2), _f32)] +
                  [jax.ShapeDtypeStruct((_NP, 1), _f32)] * 3,
    )(a0, a1, hp1, ws1, b1, w2, asr, adr)


def _tc3(a0, a1, hp2, ws2, b2):
    return pl.pallas_call(
        _tc3_body,
        grid=(_GRID,),
        in_specs=[_row_block(_W2), _row_block(_W2), _row_block(_W2),
                  _row_block(1), _full_block((1, _NCLS))],
        out_specs=_row_block(_NCLS),
        out_shape=jax.ShapeDtypeStruct((_NP, _NCLS), _f32),
    )(a0, a1, hp2, ws2, b2)


def _sc_body(width, ebuf_hbm, as_hbm, ad_hbm, hp_hbm, out_hbm,
             as_v, ad_v, eb0, eb1, db0, db1, rows0, rows1, wbuf, zbuf, acc,
             semi0, semi1, semg0, semg1, sems0, sems1):
    c = lax.axis_index("c")
    s = lax.axis_index("s")
    wid = s * 2 + c
    cbase = wid * _CPT
    eb = (eb0, eb1)
    db = (db0, db1)
    rows = (rows0, rows1)
    semi = (semi0, semi1)
    semg = (semg0, semg1)
    sems = (sems0, sems1)

    def gather_start(i, r):
        for h in range(_SUB):
            pltpu.async_copy(hp_hbm.at[eb[i].at[h]],
                             rows[i].at[pl.ds(h * _CH, _CH)], semg[i])

    def gather_wait(i):
        for h in range(_SUB):
            pltpu.make_async_copy(hp_hbm.at[eb[i].at[h]],
                                  rows[i].at[pl.ds(h * _CH, _CH)],
                                  semg[i]).wait()

    def scatter_start(i):
        for h in range(_SUB):
            pltpu.async_copy(rows[i].at[pl.ds(h * _CH, _CH)],
                             acc.at[db[i].at[h]], sems[i], add=True)

    def scatter_wait(i):
        for h in range(_SUB):
            pltpu.make_async_copy(rows[i].at[pl.ds(h * _CH, _CH)],
                                  acc.at[db[i].at[h]], sems[i]).wait()

    pltpu.sync_copy(as_hbm, as_v)
    pltpu.sync_copy(ad_hbm, ad_v)

    zoffs = [j * 16 for j in range(width // 16)]
    if width % 16:
        zoffs.append(width - 16)

    def zb(i, carry):
        for o in zoffs:
            zbuf[i, pl.ds(o, 16)] = jnp.zeros((16,), _f32)
        return carry
    lax.fori_loop(0, 64, zb, 0)

    def za(i, carry):
        pltpu.sync_copy(zbuf, acc.at[pl.ds(s * _RPT + i * 64, 64)])
        return carry
    lax.fori_loop(0, _RPT // 64, za, 0)
    plsc.subcore_barrier()

    pltpu.async_copy(ebuf_hbm.at[cbase], eb0, semi0)
    pltpu.async_copy(ebuf_hbm.at[cbase + 1], eb1, semi1)
    pltpu.make_async_copy(ebuf_hbm.at[cbase], eb0, semi0).wait()
    gather_start(0, 0)

    def pair(g2, carry):
        for b in range(2):
            g = g2 * 2 + b
            nb = 1 - b
            gather_wait(b)
            @pl.when(g >= 1)
            def _():
                scatter_wait(nb)
            @pl.when(g + 1 < _CPT)
            def _():
                pltpu.make_async_copy(ebuf_hbm.at[cbase + g + 1], eb[nb],
                                      semi[nb]).wait()
                gather_start(nb, 0)
            for kk in range(_CE // 16):
                h, o = kk // (_CH // 16), (kk % (_CH // 16)) * 16
                si = eb[b][h, pl.ds(o, 16)]
                di = eb[b][_SUB + h, pl.ds(o, 16)]
                db[b][h, pl.ds(o, 16)] = di
                a = (plsc.load_gather(as_v, [si]) +
                     plsc.load_gather(ad_v, [di]))
                wbuf[pl.ds(kk * 16, 16)] = jnp.exp(jnp.maximum(a, 0.2 * a))
            nfull = width // 16
            tail = width % 16
            tail_mask = lax.iota(jnp.int32, 16) < (16 - tail)

            def scale(k, cy):
                wv = plsc.load_gather(wbuf, [jnp.broadcast_to(k, (16,))])
                for j in range(nfull):
                    rows[b][k, pl.ds(j * 16, 16)] = (
                        rows[b][k, pl.ds(j * 16, 16)] * wv)
                if tail:
                    wt = jnp.where(tail_mask, jnp.ones((16,), _f32), wv)
                    rows[b][k, pl.ds(width - 16, 16)] = (
                        rows[b][k, pl.ds(width - 16, 16)] * wt)
                return cy
            lax.fori_loop(0, _CE, scale, 0, unroll=4)
            scatter_start(b)
            @pl.when(g + 2 < _CPT)
            def _():
                pltpu.async_copy(ebuf_hbm.at[cbase + g + 2], eb[b], semi[b])
        return carry
    lax.fori_loop(0, _CPT // 2, pair, 0)
    scatter_wait((_CPT - 1) % 2)
    plsc.subcore_barrier()

    def cp(i, carry):
        r0 = s * _RPT + i * 64
        pltpu.sync_copy(acc.at[pl.ds(r0, 64)],
                        out_hbm.at[pl.ds(c * _NP + r0, 64)])
        return carry
    lax.fori_loop(0, _RPT // 64, cp, 0)


@functools.lru_cache(maxsize=None)
def _make_sc_agg(width):
    @functools.partial(
        pl.kernel,
        mesh=plsc.VectorSubcoreMesh(core_axis_name="c", subcore_axis_name="s"),
        out_type=jax.ShapeDtypeStruct((2 * _NP, width), _f32),
        compiler_params=pltpu.CompilerParams(needs_layout_passes=False,
                                             use_tc_tiling_on_sc=False),
        scratch_types=[
            pltpu.VMEM((_NP,), _f32),
            pltpu.VMEM((_NP,), _f32),
            pltpu.VMEM((2 * _SUB, _CH), jnp.int32),
            pltpu.VMEM((2 * _SUB, _CH), jnp.int32),
            pltpu.VMEM((_SUB, _CH), jnp.int32),
            pltpu.VMEM((_SUB, _CH), jnp.int32),
            pltpu.VMEM((_CE, width), _f32),
            pltpu.VMEM((_CE, width), _f32),
            pltpu.VMEM((_CE,), _f32),
            pltpu.VMEM((64, width), _f32),
            pltpu.VMEM_SHARED((_NP, width), _f32),
            pltpu.SemaphoreType.DMA,
            pltpu.SemaphoreType.DMA,
            pltpu.SemaphoreType.DMA,
            pltpu.SemaphoreType.DMA,
            pltpu.SemaphoreType.DMA,
            pltpu.SemaphoreType.DMA,
        ],
    )
    def _sc_agg(ebuf_hbm, as_hbm, ad_hbm, hp_hbm, out_hbm, *scratch):
        _sc_body(width, ebuf_hbm, as_hbm, ad_hbm, hp_hbm, out_hbm, *scratch)

    return _sc_agg


def kernel(x, edge_index, W1, att_src1, att_dst1, b1, W2, att_src2, att_dst2,
           b2):
    xp = jnp.zeros((_NP, _DIN), _f32).at[:_N].set(x)
    src = jnp.full((_EP,), 0, jnp.int32).at[:_E].set(
        edge_index[0].astype(jnp.int32))
    dst = (_JUNK + jnp.arange(_EP, dtype=jnp.int32) % 224).at[:_E].set(
        edge_index[1].astype(jnp.int32))
    ebuf = jnp.concatenate(
        [src.reshape(_NTILES * _CPT, _SUB, _CH),
         dst.reshape(_NTILES * _CPT, _SUB, _CH)], axis=1)
    ebuf = (ebuf.reshape(_CPT, _NTILES, 2 * _SUB, _CH)
            .transpose(1, 0, 2, 3).reshape(_NTILES * _CPT, 2 * _SUB, _CH))

    asr1 = att_src1.reshape(1, _HID)
    adr1 = att_dst1.reshape(1, _HID)
    asr2 = att_src2.reshape(1, _NCLS)
    adr2 = att_dst2.reshape(1, _NCLS)

    hp1, als1, ald1, ws1 = _tc1(xp, W1, asr1, adr1)
    acc1 = _make_sc_agg(_W1)(ebuf, als1.reshape(_NP), ald1.reshape(_NP), hp1)
    hp2, als2, ald2, ws2 = _tc2(acc1[:_NP], acc1[_NP:], hp1, ws1,
                                b1.reshape(1, _HID), W2, asr2, adr2)
    acc2 = _make_sc_agg(_W2)(ebuf, als2.reshape(_NP), ald2.reshape(_NP), hp2)
    out = _tc3(acc2[:_NP], acc2[_NP:], hp2, ws2, b2.reshape(1, _NCLS))
    return out[:_N]
